# baseline (speedup 1.0000x reference)
import functools

import numpy as np

import jax
import jax.numpy as jnp
from jax import lax
from jax.experimental import pallas as pl
from jax.experimental.pallas import tpu as pltpu
from jax.experimental.pallas import tpu_sc as plsc

S, D, E, K, NS = 2048, 768, 8, 2, 2
H = 4 * D
T = 128
NB = 39
CAP = NB * T
NTILES = 32
PPT = (K * S) // NTILES
TPC = 32


_INV_SQRT2 = np.float32(1.0 / np.sqrt(2.0))


def _gelu(v):
    return 0.5 * v * (1.0 + lax.erf(v * _INV_SQRT2))


def _router_body(x_ref, cent_ref, bias_ref, pos_ref, w1_ref, w2_ref, meta_ref):
    x = x_ref[...]
    raw = lax.dot_general(x, cent_ref[...], (((1,), (1,)), ((), ())),
                          preferred_element_type=jnp.float32)
    sig = jax.nn.sigmoid(raw)
    bs = jax.nn.sigmoid(raw + bias_ref[...])
    eio = lax.broadcasted_iota(jnp.int32, (S, E), 1)
    m1 = jnp.max(bs, axis=1, keepdims=True)
    a1 = jnp.min(jnp.where(bs >= m1, eio, E), axis=1, keepdims=True)
    bs2 = jnp.where(eio == a1, -jnp.inf, bs)
    m2 = jnp.max(bs2, axis=1, keepdims=True)
    a2 = jnp.min(jnp.where(bs2 >= m2, eio, E), axis=1, keepdims=True)
    g1 = jnp.sum(jnp.where(eio == a1, sig, 0.0), axis=1, keepdims=True)
    g2 = jnp.sum(jnp.where(eio == a2, sig, 0.0), axis=1, keepdims=True)
    mx = jnp.maximum(g1, g2)
    e1 = jnp.exp(g1 - mx)
    e2 = jnp.exp(g2 - mx)
    den = e1 + e2
    ones16 = jnp.ones((1, 16), jnp.float32)
    w1_ref[...] = (e1 / den) * ones16
    w2_ref[...] = (e2 / den) * ones16

    rs = (eio == a1).astype(jnp.float32) + (eio == a2).astype(jnp.float32)
    pref = rs
    sh = 1
    while sh < S:
        pref = pref + jnp.concatenate(
            [jnp.zeros((sh, E), jnp.float32), pref[:S - sh]], axis=0)
        sh *= 2
    pref = pref - rs
    counts = jnp.sum(rs, axis=0, keepdims=True)
    pc = jnp.ceil(counts / T) * T
    off_list = []
    acc = jnp.zeros((1, 1), jnp.float32)
    for e in range(E):
        off_list.append(acc)
        acc = acc + pc[0:1, e:e + 1]
    total = acc
    rank0 = jnp.zeros((S, 1), jnp.float32)
    rank1 = jnp.zeros((S, 1), jnp.float32)
    off0 = jnp.zeros((S, 1), jnp.float32)
    off1 = jnp.zeros((S, 1), jnp.float32)
    for e in range(E):
        pe = pref[:, e:e + 1]
        rank0 = jnp.where(a1 == e, pe, rank0)
        rank1 = jnp.where(a2 == e, pe, rank1)
        off0 = jnp.where(a1 == e, off_list[e], off0)
        off1 = jnp.where(a2 == e, off_list[e], off1)
    pos0 = (off0 + rank0).astype(jnp.int32)
    pos1 = (off1 + rank1).astype(jnp.int32)
    pos_ref[...] = jnp.concatenate([pos0, pos1], axis=1)

    last_e = jnp.zeros((1, 1), jnp.float32)
    for e in range(E):
        last_e = jnp.where(pc[0:1, e:e + 1] > 0, float(e), last_e)
    blk_t = lax.broadcasted_iota(jnp.int32, (1, 128), 1).astype(jnp.float32) * T
    eob = jnp.broadcast_to(last_e, (1, 128))
    for e in range(E):
        inr = (blk_t >= off_list[e]) & (blk_t < off_list[e] + pc[0:1, e:e + 1])
        eob = jnp.where(inr, float(e), eob)
    nact = total / T

    prev = jnp.concatenate([jnp.full((1, 1), -1.0, jnp.float32),
                            eob[:, :127]], axis=1)
    rstart = (eob != prev).astype(jnp.float32)
    ridx = rstart
    sh = 1
    while sh < 128:
        ridx = ridx + jnp.concatenate(
            [jnp.zeros((1, sh), jnp.float32), ridx[:, :128 - sh]], axis=1)
        sh *= 2
    par = lax.rem(ridx.astype(jnp.int32) - 1, 2).astype(jnp.float32)
    nxte = eob
    for e in range(E - 1, -1, -1):
        sel = (pc[0:1, e:e + 1] > 0) & (eob < float(e))
        nxte = jnp.where(sel, float(e), nxte)

    rowio = lax.broadcasted_iota(jnp.int32, (8, 128), 0)
    meta = jnp.broadcast_to(nact, (8, 128))
    meta = jnp.where(rowio == 0, jnp.broadcast_to(eob, (8, 128)), meta)
    meta = jnp.where(rowio == 2, jnp.broadcast_to(rstart, (8, 128)), meta)
    meta = jnp.where(rowio == 3, jnp.broadcast_to(nxte, (8, 128)), meta)
    meta = jnp.where(rowio == 4, jnp.broadcast_to(par, (8, 128)), meta)
    meta_ref[...] = meta.astype(jnp.int32)


def _router(x2, cent, bias2):
    return pl.pallas_call(
        _router_body,
        out_shape=[
            jax.ShapeDtypeStruct((S, K), jnp.int32),
            jax.ShapeDtypeStruct((S, 16), jnp.float32),
            jax.ShapeDtypeStruct((S, 16), jnp.float32),
            jax.ShapeDtypeStruct((8, 128), jnp.int32),
        ],
    )(x2, cent, bias2)


def _shared_body(x_ref, fc_hbm, pj_hbm, out_ref, wfc, wpj, sems):
    i = pl.program_id(0)

    def cp(hbm, dst, s, b):
        return pltpu.make_async_copy(hbm.at[s], dst.at[s], sems.at[b])

    @pl.when(i == 0)
    def _():
        cp(fc_hbm, wfc, 0, 0).start()
        cp(pj_hbm, wpj, 0, 1).start()
        cp(fc_hbm, wfc, 1, 2).start()
        cp(pj_hbm, wpj, 1, 3).start()
        cp(fc_hbm, wfc, 0, 0).wait()

    xb = x_ref[...]
    h = lax.dot_general(xb, wfc[0], (((1,), (1,)), ((), ())),
                        preferred_element_type=jnp.float32)
    h = _gelu(h)

    @pl.when(i == 0)
    def _():
        cp(pj_hbm, wpj, 0, 1).wait()

    acc = xb + lax.dot_general(h, wpj[0], (((1,), (1,)), ((), ())),
                               preferred_element_type=jnp.float32)

    @pl.when(i == 0)
    def _():
        cp(fc_hbm, wfc, 1, 2).wait()

    h = lax.dot_general(xb, wfc[1], (((1,), (1,)), ((), ())),
                        preferred_element_type=jnp.float32)
    h = _gelu(h)

    @pl.when(i == 0)
    def _():
        cp(pj_hbm, wpj, 1, 3).wait()

    out_ref[...] = acc + lax.dot_general(h, wpj[1], (((1,), (1,)), ((), ())),
                                         preferred_element_type=jnp.float32)


def _shared(x2, sfc, sproj):
    ts = 256
    return pl.pallas_call(
        _shared_body,
        grid=(S // ts,),
        in_specs=[
            pl.BlockSpec((ts, D), lambda i: (i, 0)),
            pl.BlockSpec(memory_space=pl.ANY),
            pl.BlockSpec(memory_space=pl.ANY),
        ],
        out_specs=pl.BlockSpec((ts, D), lambda i: (i, 0)),
        out_shape=jax.ShapeDtypeStruct((S, D), jnp.float32),
        scratch_shapes=[
            pltpu.VMEM((NS, H, D), jnp.float32),
            pltpu.VMEM((NS, D, H), jnp.float32),
            pltpu.SemaphoreType.DMA((4,)),
        ],
    )(x2, sfc, sproj)


def _routed_body(eob_r, na_r, rst_r, nxt_r, par_r, xs_ref, rfc_hbm, rproj_hbm,
                 ys_ref, wfc, wpj, sem_fc, sem_pj):
    i = pl.program_id(0)
    e_i = eob_r[i]
    p_i = par_r[i]

    def fc_copy(e, b):
        return pltpu.make_async_copy(rfc_hbm.at[e], wfc.at[b], sem_fc)

    def pj_copy(e, b):
        return pltpu.make_async_copy(rproj_hbm.at[e], wpj.at[b], sem_pj)

    @pl.when(i == 0)
    def _():
        fc_copy(e_i, p_i).start()
        pj_copy(e_i, p_i).start()
        fc_copy(e_i, p_i).wait()
        pj_copy(e_i, p_i).wait()

    @pl.when(jnp.logical_and(i > 0, rst_r[i] == 1))
    def _():
        fc_copy(e_i, p_i).wait()
        pj_copy(e_i, p_i).wait()

    @pl.when(jnp.logical_and(rst_r[i] == 1, nxt_r[i] != e_i))
    def _():
        fc_copy(nxt_r[i], 1 - p_i).start()
        pj_copy(nxt_r[i], 1 - p_i).start()

    @pl.when(i < na_r[0])
    def _():
        xb = xs_ref[...]
        h = lax.dot_general(xb, wfc[p_i], (((1,), (1,)), ((), ())),
                            preferred_element_type=jnp.float32)
        h = _gelu(h)
        ys_ref[...] = lax.dot_general(h, wpj[p_i], (((1,), (1,)), ((), ())),
                                      preferred_element_type=jnp.float32)


def _routed(xs, rfc, rproj, eob, nact, rst, nxt, par):
    grid_spec = pltpu.PrefetchScalarGridSpec(
        num_scalar_prefetch=5,
        grid=(NB,),
        in_specs=[
            pl.BlockSpec((T, D), lambda i, *s: (i, 0)),
            pl.BlockSpec(memory_space=pl.ANY),
            pl.BlockSpec(memory_space=pl.ANY),
        ],
        out_specs=pl.BlockSpec((T, D), lambda i, *s: (i, 0)),
        scratch_shapes=[
            pltpu.VMEM((2, H, D), jnp.float32),
            pltpu.VMEM((2, D, H), jnp.float32),
            pltpu.SemaphoreType.DMA,
            pltpu.SemaphoreType.DMA,
        ],
    )
    return pl.pallas_call(
        _routed_body,
        grid_spec=grid_spec,
        out_shape=jax.ShapeDtypeStruct((CAP, D), jnp.float32),
    )(eob, nact, rst, nxt, par, xs, rfc, rproj)


@functools.cache
def _sc_mesh():
    return plsc.VectorSubcoreMesh(core_axis_name="c", subcore_axis_name="s")


_NCHK = 4
_CSZ = PPT // _NCHK


def _sc_dispatch_body(x_hbm, tok_hbm, pos_hbm, xs_hbm, *scratch):
    tok_c = scratch[0:_NCHK]
    pos_c = scratch[_NCHK:2 * _NCHK]
    rows_c = scratch[2 * _NCHK:3 * _NCHK]
    gsems = scratch[3 * _NCHK:4 * _NCHK]
    ssem = scratch[4 * _NCHK]
    wid = lax.axis_index("s") * 2 + lax.axis_index("c")
    base = wid * PPT
    for c in range(_NCHK):
        pltpu.sync_copy(tok_hbm.at[pl.ds(base + c * _CSZ, _CSZ)], tok_c[c])
        pltpu.sync_copy(pos_hbm.at[pl.ds(base + c * _CSZ, _CSZ)], pos_c[c])
    gd = [None] * _NCHK
    sd = [None] * _NCHK
    gd[0] = pltpu.async_copy(x_hbm.at[tok_c[0]], rows_c[0], gsems[0])
    for c in range(_NCHK):
        if c + 1 < _NCHK:
            gd[c + 1] = pltpu.async_copy(x_hbm.at[tok_c[c + 1]],
                                         rows_c[c + 1], gsems[c + 1])
        gd[c].wait()
        sd[c] = pltpu.async_copy(rows_c[c], xs_hbm.at[pos_c[c]], ssem)
    for c in range(_NCHK):
        sd[c].wait()


@functools.cache
def _sc_dispatch():
    return pl.kernel(
        _sc_dispatch_body,
        out_type=jax.ShapeDtypeStruct((CAP, D), jnp.float32),
        mesh=_sc_mesh(),
        scratch_types=(
            [pltpu.VMEM((_CSZ,), jnp.int32) for _ in range(_NCHK)]
            + [pltpu.VMEM((_CSZ,), jnp.int32) for _ in range(_NCHK)]
            + [pltpu.VMEM((_CSZ, D), jnp.float32) for _ in range(_NCHK)]
            + [pltpu.SemaphoreType.DMA for _ in range(_NCHK)]
            + [pltpu.SemaphoreType.DMA]
        ),
    )


_QN = 4
_QSZ = (S // NTILES) // _QN


def _sc_combine_body(base_hbm, ys_hbm, pos1_hbm, pos2_hbm, w1_hbm, w2_hbm,
                     out_hbm, *scratch):
    p1 = scratch[0:2]
    p2 = scratch[2:4]
    r1 = scratch[4:6]
    r2 = scratch[6:8]
    bb = scratch[8:10]
    w1 = scratch[10:12]
    w2 = scratch[12:14]
    s1 = scratch[14:16]
    s2 = scratch[16:18]
    s3 = scratch[18:20]
    ws = scratch[20:22]
    wid = lax.axis_index("s") * 2 + lax.axis_index("c")
    t00 = wid * (S // NTILES)

    def issue(q, st):
        t0 = t00 + q * _QSZ
        pltpu.sync_copy(pos1_hbm.at[pl.ds(t0, _QSZ)], p1[st])
        pltpu.sync_copy(pos2_hbm.at[pl.ds(t0, _QSZ)], p2[st])
        pltpu.sync_copy(w1_hbm.at[pl.ds(t0, _QSZ)], w1[st])
        pltpu.sync_copy(w2_hbm.at[pl.ds(t0, _QSZ)], w2[st])
        d1 = pltpu.async_copy(ys_hbm.at[p1[st]], r1[st], s1[st])
        d2 = pltpu.async_copy(ys_hbm.at[p2[st]], r2[st], s2[st])
        db = pltpu.async_copy(base_hbm.at[pl.ds(t0, _QSZ)], bb[st], s3[st])
        return (d1, d2, db)

    ds = {0: issue(0, 0)}
    wd = {}
    for q in range(_QN):
        st = q % 2
        if q + 1 < _QN:
            if (q - 1) in wd:
                wd[q - 1].wait()
            ds[q + 1] = issue(q + 1, (q + 1) % 2)
        for d in ds[q]:
            d.wait()

        def body(j, carry):
            wv1 = w1[st][j, :]
            wv2 = w2[st][j, :]
            for t_ in range(D // 16):
                sl = pl.ds(t_ * 16, 16)
                bb[st][j, sl] = (bb[st][j, sl] + wv1 * r1[st][j, sl]
                                 + wv2 * r2[st][j, sl])
            return carry

        lax.fori_loop(0, _QSZ, body, 0)
        wd[q] = pltpu.async_copy(bb[st], out_hbm.at[pl.ds(t00 + q * _QSZ,
                                                          _QSZ)], ws[st])
    wd[_QN - 2].wait()
    wd[_QN - 1].wait()


@functools.cache
def _sc_combine():
    return pl.kernel(
        _sc_combine_body,
        out_type=jax.ShapeDtypeStruct((S, D), jnp.float32),
        mesh=_sc_mesh(),
        scratch_types=(
            [pltpu.VMEM((_QSZ,), jnp.int32) for _ in range(4)]
            + [pltpu.VMEM((_QSZ, D), jnp.float32) for _ in range(6)]
            + [pltpu.VMEM((_QSZ, 16), jnp.float32) for _ in range(4)]
            + [pltpu.SemaphoreType.DMA for _ in range(8)]
        ),
    )


def kernel(x, shared_fc, shared_proj, routed_fc, routed_proj, centroids,
           routing_biases):
    x2 = x.reshape(S, D)
    bias2 = routing_biases.reshape(1, E)
    pos, w1r, w2r, meta = _router(x2, centroids, bias2)
    eob = meta[0, :NB]
    nact = meta[1, 0:1]
    rst = meta[2, :NB]
    nxt = meta[3, :NB]
    par = meta[4, :NB]
    tok_flat = jnp.repeat(jnp.arange(S, dtype=jnp.int32), K)
    pos_flat = pos.reshape(K * S)
    xs = _sc_dispatch()(x2, tok_flat, pos_flat)
    base = _shared(x2, shared_fc, shared_proj)
    ys = _routed(xs, routed_fc, routed_proj, eob, nact, rst, nxt, par)
    out = _sc_combine()(base, ys, pos[:, 0], pos[:, 1], w1r, w2r)
    return out.reshape(1, S, D)


# speedup vs baseline: 1.0574x; 1.0574x over previous
import functools

import numpy as np

import jax
import jax.numpy as jnp
from jax import lax
from jax.experimental import pallas as pl
from jax.experimental.pallas import tpu as pltpu
from jax.experimental.pallas import tpu_sc as plsc

S, D, E, K, NS = 2048, 768, 8, 2, 2
H = 4 * D
T = 128
NB = 39
CAP = NB * T
NTILES = 32
PPT = (K * S) // NTILES
TPC = 32


_INV_SQRT2 = np.float32(1.0 / np.sqrt(2.0))


def _gelu(v):
    return 0.5 * v * (1.0 + lax.erf(v * _INV_SQRT2))


def _router_body(x_ref, cent_ref, bias_ref, pos_ref, w1_ref, w2_ref, meta_ref):
    x = x_ref[...]
    raw = lax.dot_general(x, cent_ref[...], (((1,), (1,)), ((), ())),
                          preferred_element_type=jnp.float32)
    sig = jax.nn.sigmoid(raw)
    bs = jax.nn.sigmoid(raw + bias_ref[...])
    eio = lax.broadcasted_iota(jnp.int32, (S, E), 1)
    m1 = jnp.max(bs, axis=1, keepdims=True)
    a1 = jnp.min(jnp.where(bs >= m1, eio, E), axis=1, keepdims=True)
    bs2 = jnp.where(eio == a1, -jnp.inf, bs)
    m2 = jnp.max(bs2, axis=1, keepdims=True)
    a2 = jnp.min(jnp.where(bs2 >= m2, eio, E), axis=1, keepdims=True)
    g1 = jnp.sum(jnp.where(eio == a1, sig, 0.0), axis=1, keepdims=True)
    g2 = jnp.sum(jnp.where(eio == a2, sig, 0.0), axis=1, keepdims=True)
    mx = jnp.maximum(g1, g2)
    e1 = jnp.exp(g1 - mx)
    e2 = jnp.exp(g2 - mx)
    den = e1 + e2
    ones16 = jnp.ones((1, 16), jnp.float32)
    w1_ref[...] = (e1 / den) * ones16
    w2_ref[...] = (e2 / den) * ones16

    rs = (eio == a1).astype(jnp.float32) + (eio == a2).astype(jnp.float32)
    pref = rs
    sh = 1
    while sh < S:
        pref = pref + jnp.concatenate(
            [jnp.zeros((sh, E), jnp.float32), pref[:S - sh]], axis=0)
        sh *= 2
    pref = pref - rs
    counts = jnp.sum(rs, axis=0, keepdims=True)
    pc = jnp.ceil(counts / T) * T
    off_list = []
    acc = jnp.zeros((1, 1), jnp.float32)
    for e in range(E):
        off_list.append(acc)
        acc = acc + pc[0:1, e:e + 1]
    total = acc
    rank0 = jnp.zeros((S, 1), jnp.float32)
    rank1 = jnp.zeros((S, 1), jnp.float32)
    off0 = jnp.zeros((S, 1), jnp.float32)
    off1 = jnp.zeros((S, 1), jnp.float32)
    for e in range(E):
        pe = pref[:, e:e + 1]
        rank0 = jnp.where(a1 == e, pe, rank0)
        rank1 = jnp.where(a2 == e, pe, rank1)
        off0 = jnp.where(a1 == e, off_list[e], off0)
        off1 = jnp.where(a2 == e, off_list[e], off1)
    pos0 = (off0 + rank0).astype(jnp.int32)
    pos1 = (off1 + rank1).astype(jnp.int32)
    pos_ref[...] = jnp.concatenate([pos0, pos1], axis=1)

    last_e = jnp.zeros((1, 1), jnp.float32)
    for e in range(E):
        last_e = jnp.where(pc[0:1, e:e + 1] > 0, float(e), last_e)
    blk_t = lax.broadcasted_iota(jnp.int32, (1, 128), 1).astype(jnp.float32) * T
    eob = jnp.broadcast_to(last_e, (1, 128))
    for e in range(E):
        inr = (blk_t >= off_list[e]) & (blk_t < off_list[e] + pc[0:1, e:e + 1])
        eob = jnp.where(inr, float(e), eob)
    nact = total / T

    prev = jnp.concatenate([jnp.full((1, 1), -1.0, jnp.float32),
                            eob[:, :127]], axis=1)
    rstart = (eob != prev).astype(jnp.float32)
    ridx = rstart
    sh = 1
    while sh < 128:
        ridx = ridx + jnp.concatenate(
            [jnp.zeros((1, sh), jnp.float32), ridx[:, :128 - sh]], axis=1)
        sh *= 2
    par = lax.rem(ridx.astype(jnp.int32) - 1, 2).astype(jnp.float32)
    nxte = eob
    for e in range(E - 1, -1, -1):
        sel = (pc[0:1, e:e + 1] > 0) & (eob < float(e))
        nxte = jnp.where(sel, float(e), nxte)

    rowio = lax.broadcasted_iota(jnp.int32, (8, 128), 0)
    meta = jnp.broadcast_to(nact, (8, 128))
    meta = jnp.where(rowio == 0, jnp.broadcast_to(eob, (8, 128)), meta)
    meta = jnp.where(rowio == 2, jnp.broadcast_to(rstart, (8, 128)), meta)
    meta = jnp.where(rowio == 3, jnp.broadcast_to(nxte, (8, 128)), meta)
    meta = jnp.where(rowio == 4, jnp.broadcast_to(par, (8, 128)), meta)
    meta_ref[...] = meta.astype(jnp.int32)


def _router(x2, cent, bias2):
    return pl.pallas_call(
        _router_body,
        out_shape=[
            jax.ShapeDtypeStruct((S, K), jnp.int32),
            jax.ShapeDtypeStruct((S, 16), jnp.float32),
            jax.ShapeDtypeStruct((S, 16), jnp.float32),
            jax.ShapeDtypeStruct((8, 128), jnp.int32),
        ],
    )(x2, cent, bias2)


def _shared_body(x_ref, fc_ref, proj_ref, out_ref):
    xb = x_ref[...]
    acc = xb
    for s in range(NS):
        h = lax.dot_general(xb, fc_ref[s], (((1,), (1,)), ((), ())),
                            preferred_element_type=jnp.float32)
        h = _gelu(h)
        acc = acc + lax.dot_general(h, proj_ref[s], (((1,), (1,)), ((), ())),
                                    preferred_element_type=jnp.float32)
    out_ref[...] = acc


def _shared(x2, sfc, sproj):
    ts = 256
    return pl.pallas_call(
        _shared_body,
        grid=(S // ts,),
        in_specs=[
            pl.BlockSpec((ts, D), lambda i: (i, 0)),
            pl.BlockSpec((NS, H, D), lambda i: (0, 0, 0)),
            pl.BlockSpec((NS, D, H), lambda i: (0, 0, 0)),
        ],
        out_specs=pl.BlockSpec((ts, D), lambda i: (i, 0)),
        out_shape=jax.ShapeDtypeStruct((S, D), jnp.float32),
    )(x2, sfc, sproj)


def _routed_body(eob_r, na_r, rst_r, nxt_r, par_r, xs_ref, rfc_hbm, rproj_hbm,
                 ys_ref, wfc, wpj, sem_fc, sem_pj):
    i = pl.program_id(0)
    e_i = eob_r[i]
    p_i = par_r[i]

    def fc_copy(e, b):
        return pltpu.make_async_copy(rfc_hbm.at[e], wfc.at[b], sem_fc)

    def pj_copy(e, b):
        return pltpu.make_async_copy(rproj_hbm.at[e], wpj.at[b], sem_pj)

    @pl.when(i == 0)
    def _():
        fc_copy(e_i, p_i).start()
        pj_copy(e_i, p_i).start()
        fc_copy(e_i, p_i).wait()
        pj_copy(e_i, p_i).wait()

    @pl.when(jnp.logical_and(i > 0, rst_r[i] == 1))
    def _():
        fc_copy(e_i, p_i).wait()
        pj_copy(e_i, p_i).wait()

    @pl.when(jnp.logical_and(rst_r[i] == 1, nxt_r[i] != e_i))
    def _():
        fc_copy(nxt_r[i], 1 - p_i).start()
        pj_copy(nxt_r[i], 1 - p_i).start()

    @pl.when(i < na_r[0])
    def _():
        xb = xs_ref[...]
        h = lax.dot_general(xb, wfc[p_i], (((1,), (1,)), ((), ())),
                            preferred_element_type=jnp.float32)
        h = _gelu(h)
        ys_ref[...] = lax.dot_general(h, wpj[p_i], (((1,), (1,)), ((), ())),
                                      preferred_element_type=jnp.float32)


def _routed(xs, rfc, rproj, eob, nact, rst, nxt, par):
    grid_spec = pltpu.PrefetchScalarGridSpec(
        num_scalar_prefetch=5,
        grid=(NB,),
        in_specs=[
            pl.BlockSpec((T, D), lambda i, *s: (i, 0)),
            pl.BlockSpec(memory_space=pl.ANY),
            pl.BlockSpec(memory_space=pl.ANY),
        ],
        out_specs=pl.BlockSpec((T, D), lambda i, *s: (i, 0)),
        scratch_shapes=[
            pltpu.VMEM((2, H, D), jnp.float32),
            pltpu.VMEM((2, D, H), jnp.float32),
            pltpu.SemaphoreType.DMA,
            pltpu.SemaphoreType.DMA,
        ],
    )
    return pl.pallas_call(
        _routed_body,
        grid_spec=grid_spec,
        out_shape=jax.ShapeDtypeStruct((CAP, D), jnp.float32),
    )(eob, nact, rst, nxt, par, xs, rfc, rproj)


@functools.cache
def _sc_mesh():
    return plsc.VectorSubcoreMesh(core_axis_name="c", subcore_axis_name="s")


_NCHK = 4
_CSZ = PPT // _NCHK


def _sc_dispatch_body(x_hbm, tok_hbm, pos_hbm, xs_hbm, *scratch):
    tok_c = scratch[0:_NCHK]
    pos_c = scratch[_NCHK:2 * _NCHK]
    rows_c = scratch[2 * _NCHK:3 * _NCHK]
    gsems = scratch[3 * _NCHK:4 * _NCHK]
    ssem = scratch[4 * _NCHK]
    wid = lax.axis_index("s") * 2 + lax.axis_index("c")
    base = wid * PPT
    for c in range(_NCHK):
        pltpu.sync_copy(tok_hbm.at[pl.ds(base + c * _CSZ, _CSZ)], tok_c[c])
        pltpu.sync_copy(pos_hbm.at[pl.ds(base + c * _CSZ, _CSZ)], pos_c[c])
    gd = [None] * _NCHK
    sd = [None] * _NCHK
    gd[0] = pltpu.async_copy(x_hbm.at[tok_c[0]], rows_c[0], gsems[0])
    for c in range(_NCHK):
        if c + 1 < _NCHK:
            gd[c + 1] = pltpu.async_copy(x_hbm.at[tok_c[c + 1]],
                                         rows_c[c + 1], gsems[c + 1])
        gd[c].wait()
        sd[c] = pltpu.async_copy(rows_c[c], xs_hbm.at[pos_c[c]], ssem)
    for c in range(_NCHK):
        sd[c].wait()


@functools.cache
def _sc_dispatch():
    return pl.kernel(
        _sc_dispatch_body,
        out_type=jax.ShapeDtypeStruct((CAP, D), jnp.float32),
        mesh=_sc_mesh(),
        scratch_types=(
            [pltpu.VMEM((_CSZ,), jnp.int32) for _ in range(_NCHK)]
            + [pltpu.VMEM((_CSZ,), jnp.int32) for _ in range(_NCHK)]
            + [pltpu.VMEM((_CSZ, D), jnp.float32) for _ in range(_NCHK)]
            + [pltpu.SemaphoreType.DMA for _ in range(_NCHK)]
            + [pltpu.SemaphoreType.DMA]
        ),
    )


_QN = 4
_QSZ = (S // NTILES) // _QN


def _sc_combine_body(base_hbm, ys_hbm, pos1_hbm, pos2_hbm, w1_hbm, w2_hbm,
                     out_hbm, *scratch):
    p1 = scratch[0:2]
    p2 = scratch[2:4]
    r1 = scratch[4:6]
    r2 = scratch[6:8]
    bb = scratch[8:10]
    w1 = scratch[10:12]
    w2 = scratch[12:14]
    s1 = scratch[14:16]
    s2 = scratch[16:18]
    s3 = scratch[18:20]
    ws = scratch[20:22]
    wid = lax.axis_index("s") * 2 + lax.axis_index("c")
    t00 = wid * (S // NTILES)

    def issue(q, st):
        t0 = t00 + q * _QSZ
        pltpu.sync_copy(pos1_hbm.at[pl.ds(t0, _QSZ)], p1[st])
        pltpu.sync_copy(pos2_hbm.at[pl.ds(t0, _QSZ)], p2[st])
        pltpu.sync_copy(w1_hbm.at[pl.ds(t0, _QSZ)], w1[st])
        pltpu.sync_copy(w2_hbm.at[pl.ds(t0, _QSZ)], w2[st])
        d1 = pltpu.async_copy(ys_hbm.at[p1[st]], r1[st], s1[st])
        d2 = pltpu.async_copy(ys_hbm.at[p2[st]], r2[st], s2[st])
        db = pltpu.async_copy(base_hbm.at[pl.ds(t0, _QSZ)], bb[st], s3[st])
        return (d1, d2, db)

    ds = {0: issue(0, 0)}
    wd = {}
    for q in range(_QN):
        st = q % 2
        if q + 1 < _QN:
            if (q - 1) in wd:
                wd[q - 1].wait()
            ds[q + 1] = issue(q + 1, (q + 1) % 2)
        for d in ds[q]:
            d.wait()

        def body(j, carry):
            wv1 = w1[st][j, :]
            wv2 = w2[st][j, :]
            for t_ in range(D // 16):
                sl = pl.ds(t_ * 16, 16)
                bb[st][j, sl] = (bb[st][j, sl] + wv1 * r1[st][j, sl]
                                 + wv2 * r2[st][j, sl])
            return carry

        lax.fori_loop(0, _QSZ, body, 0)
        wd[q] = pltpu.async_copy(bb[st], out_hbm.at[pl.ds(t00 + q * _QSZ,
                                                          _QSZ)], ws[st])
    wd[_QN - 2].wait()
    wd[_QN - 1].wait()


@functools.cache
def _sc_combine():
    return pl.kernel(
        _sc_combine_body,
        out_type=jax.ShapeDtypeStruct((S, D), jnp.float32),
        mesh=_sc_mesh(),
        scratch_types=(
            [pltpu.VMEM((_QSZ,), jnp.int32) for _ in range(4)]
            + [pltpu.VMEM((_QSZ, D), jnp.float32) for _ in range(6)]
            + [pltpu.VMEM((_QSZ, 16), jnp.float32) for _ in range(4)]
            + [pltpu.SemaphoreType.DMA for _ in range(8)]
        ),
    )


def kernel(x, shared_fc, shared_proj, routed_fc, routed_proj, centroids,
           routing_biases):
    x2 = x.reshape(S, D)
    bias2 = routing_biases.reshape(1, E)
    pos, w1r, w2r, meta = _router(x2, centroids, bias2)
    eob = meta[0, :NB]
    nact = meta[1, 0:1]
    rst = meta[2, :NB]
    nxt = meta[3, :NB]
    par = meta[4, :NB]
    tok_flat = jnp.repeat(jnp.arange(S, dtype=jnp.int32), K)
    pos_flat = pos.reshape(K * S)
    xs = _sc_dispatch()(x2, tok_flat, pos_flat)
    base = _shared(x2, shared_fc, shared_proj)
    ys = _routed(xs, routed_fc, routed_proj, eob, nact, rst, nxt, par)
    out = _sc_combine()(base, ys, pos[:, 0], pos[:, 1], w1r, w2r)
    return out.reshape(1, S, D)
